# baseline (speedup 1.0000x reference)
import functools

import jax
import jax.numpy as jnp
from jax.experimental import pallas as pl
from jax.experimental.pallas import tpu as pltpu
from jax.experimental.pallas import tpu_sc as plsc

_NC = 2
_NS = 16


def _pick_edge_window(e):
    for w in (80, 64, 40, 32, 16, 8):
        if e % (_NC * _NS * w) == 0:
            return w, 0
    w = 64
    epad = ((e + _NC * _NS * w - 1) // (_NC * _NS * w)) * (_NC * _NS * w)
    return w, epad - e


def _pick_deg_window(e_padded):
    for w in (200, 160, 128, 80, 64, 40, 32, 16, 8):
        if e_padded % (_NC * _NS * w) == 0:
            return w
    return 8


def _zero_split(nacc):
    for k in range(_NS, 0, -1):
        if nacc % k == 0 and (nacc // k) % 8 == 0:
            return k, nacc // k
    return 1, nacc


_NBUF = 4


def _sc_degree(dst, ones_w, zeros_acc):
    (e,) = dst.shape
    nacc = zeros_acc.shape[0]
    w = ones_w.shape[0]
    chunk = e // (_NC * _NS)
    nwin = chunk // w
    assert nwin >= _NBUF
    nz, rows_per = _zero_split(nacc)

    ni = 6
    assert nwin >= ni

    @functools.partial(
        pl.kernel,
        out_type=jax.ShapeDtypeStruct((_NC, nacc, 128), jnp.float32),
        mesh=plsc.VectorSubcoreMesh(core_axis_name="c", subcore_axis_name="s"),
        scratch_types=[
            [pltpu.VMEM((w,), jnp.int32) for _ in range(ni)],
            pltpu.VMEM((w, 128), jnp.float32),
            pltpu.VMEM_SHARED((nacc, 128), jnp.float32),
            [pltpu.SemaphoreType.DMA for _ in range(ni)],
            [pltpu.SemaphoreType.DMA for _ in range(ni)],
        ],
    )
    def deg_kernel(dst_hbm, ones_hbm, z_hbm, out_hbm, dbuf, ones_v, acc, isem, ssem):
        cid = jax.lax.axis_index("c")
        sid = jax.lax.axis_index("s")
        wid = sid * _NC + cid
        pltpu.sync_copy(ones_hbm, ones_v)

        @pl.when(sid < nz)
        def _():
            pltpu.sync_copy(
                z_hbm.at[pl.ds(sid * rows_per, rows_per)],
                acc.at[pl.ds(sid * rows_per, rows_per)],
            )

        plsc.subcore_barrier()
        base0 = wid * chunk

        def issue_idx(j, b):
            pltpu.async_copy(dst_hbm.at[pl.ds(base0 + j * w, w)], dbuf[b], isem[b])

        def wait_idx(j, b):
            pltpu.make_async_copy(dst_hbm.at[pl.ds(base0 + j * w, w)], dbuf[b], isem[b]).wait()

        def issue_scatter(b):
            pltpu.async_copy(ones_v, acc.at[dbuf[b]], ssem[b], add=True)

        def wait_scatter(b):
            pltpu.make_async_copy(ones_v, acc.at[dbuf[b]], ssem[b]).wait()

        for b in range(ni - 1):
            issue_idx(b, b)

        nouter = -(-nwin // ni) * ni

        @pl.loop(0, nouter, step=ni)
        def _(j0):
            for u in range(ni):
                j = j0 + u

                @pl.when(j < nwin)
                def _():
                    wait_idx(j, u)
                    issue_scatter(u)

                @pl.when(j + ni - 1 < nwin)
                def _():
                    u5 = (u + ni - 1) % ni

                    @pl.when(j >= 1)
                    def _():
                        wait_scatter(u5)

                    issue_idx(j + ni - 1, u5)

        for k in range(nwin - ni, nwin):
            wait_scatter(k % ni)

        plsc.subcore_barrier()

        @pl.when(sid < nz)
        def _():
            pltpu.sync_copy(
                acc.at[pl.ds(sid * rows_per, rows_per)],
                out_hbm.at[cid, pl.ds(sid * rows_per, rows_per)],
            )

    return deg_kernel(dst, ones_w, zeros_acc)


def _sc_edge_agg(g, src, dst, zeros_acc):
    n, h = zeros_acc.shape
    (e,) = src.shape
    w, _ = _pick_edge_window(e)
    chunk = e // (_NC * _NS)
    nwin = chunk // w
    nz, rows_per = _zero_split(n)
    half = n // 2

    ni = 2 * _NBUF
    assert nwin >= ni

    @functools.partial(
        pl.kernel,
        out_type=jax.ShapeDtypeStruct((_NC, n, h), jnp.float32),
        mesh=plsc.VectorSubcoreMesh(core_axis_name="c", subcore_axis_name="s"),
        scratch_types=[
            [pltpu.VMEM((w,), jnp.int32) for _ in range(ni)],
            [pltpu.VMEM((w,), jnp.int32) for _ in range(ni)],
            [pltpu.VMEM((w, h), jnp.float32) for _ in range(_NBUF)],
            pltpu.VMEM_SHARED((n, h), jnp.float32),
            [pltpu.SemaphoreType.DMA for _ in range(ni)],
            [pltpu.SemaphoreType.DMA for _ in range(_NBUF)],
            [pltpu.SemaphoreType.DMA for _ in range(_NBUF)],
        ],
    )
    def agg_kernel(g_hbm, src_hbm, dst_hbm, z_hbm, out_hbm, sbuf, dbuf, rbuf, acc, isem, gsem, ssem):
        cid = jax.lax.axis_index("c")
        sid = jax.lax.axis_index("s")
        wid = sid * _NC + cid

        @pl.when(sid < nz)
        def _():
            pltpu.sync_copy(
                z_hbm.at[pl.ds(sid * rows_per, rows_per)],
                acc.at[pl.ds(sid * rows_per, rows_per)],
            )

        plsc.subcore_barrier()
        base0 = wid * chunk

        def issue_idx(j, b):
            pltpu.async_copy(src_hbm.at[pl.ds(base0 + j * w, w)], sbuf[b], isem[b])
            pltpu.async_copy(dst_hbm.at[pl.ds(base0 + j * w, w)], dbuf[b], isem[b])

        def wait_idx(j, b):
            pltpu.make_async_copy(src_hbm.at[pl.ds(base0 + j * w, w)], sbuf[b], isem[b]).wait()
            pltpu.make_async_copy(dst_hbm.at[pl.ds(base0 + j * w, w)], dbuf[b], isem[b]).wait()

        def issue_gather(bi, br):
            pltpu.async_copy(g_hbm.at[sbuf[bi]], rbuf[br], gsem[br])

        def wait_gather(bi, br):
            pltpu.make_async_copy(g_hbm.at[sbuf[bi]], rbuf[br], gsem[br]).wait()

        def issue_scatter(bi, br):
            pltpu.async_copy(rbuf[br], acc.at[dbuf[bi]], ssem[br], add=True)

        def wait_scatter(bi, br):
            pltpu.make_async_copy(rbuf[br], acc.at[dbuf[bi]], ssem[br]).wait()

        lead = ni - 3
        for b in range(lead):
            issue_idx(b, b)
        for b in range(2):
            wait_idx(b, b)
            issue_gather(b, b)

        nouter = -(-nwin // ni) * ni

        @pl.loop(0, nouter, step=ni)
        def _(j0):
            for u in range(ni):
                j = j0 + u
                br = u % _NBUF

                @pl.when(j < nwin)
                def _():
                    wait_gather(u, br)
                    issue_scatter(u, br)

                @pl.when(j + 2 < nwin)
                def _():
                    u2 = (u + 2) % ni

                    @pl.when(j >= _NBUF - 2)
                    def _():
                        wait_scatter((u + ni + 2 - _NBUF) % ni, (u + 2) % _NBUF)

                    wait_idx(j + 2, u2)
                    issue_gather(u2, (u + 2) % _NBUF)

                @pl.when(j + lead < nwin)
                def _():
                    issue_idx(j + lead, (u + lead) % ni)

        for k in range(nwin - _NBUF, nwin):
            wait_scatter(k % ni, k % _NBUF)

        plsc.subcore_barrier()

        @pl.when(sid < nz)
        def _():
            pltpu.sync_copy(
                acc.at[pl.ds(sid * rows_per, rows_per)],
                out_hbm.at[cid, pl.ds(sid * rows_per, rows_per)],
            )

    return agg_kernel(g, src, dst, zeros_acc)


def _dinv_block(da, db):
    return jax.lax.rsqrt(1.0 + da[:, :1] + db[:, :1])


def _pick_node_block(n):
    for nb in (1000, 2000, 500, 1250, 200, 8):
        if n % nb == 0 and nb % 8 == 0:
            return nb
    return n


def _tc_k0(x, w0, b0, w1, b1, gw1):
    n, h = x.shape
    nb = _pick_node_block(n)

    def body(x_r, w0_r, b0_r, w1_r, b1_r, gw_r, o_r):
        a = jnp.maximum(jnp.dot(x_r[...], w0_r[...], preferred_element_type=jnp.float32) + b0_r[...], 0.0)
        h0 = jnp.dot(a, w1_r[...], preferred_element_type=jnp.float32) + b1_r[...]
        o_r[...] = jnp.dot(h0, gw_r[...], preferred_element_type=jnp.float32)

    full = lambda i: (0, 0)
    row = lambda i: (i, 0)
    return pl.pallas_call(
        body,
        grid=(n // nb,),
        in_specs=[
            pl.BlockSpec((nb, h), row),
            pl.BlockSpec((h, h), full), pl.BlockSpec((1, h), full),
            pl.BlockSpec((h, h), full), pl.BlockSpec((1, h), full),
            pl.BlockSpec((h, h), full),
        ],
        out_specs=pl.BlockSpec((nb, h), row),
        out_shape=jax.ShapeDtypeStruct((n, h), jnp.float32),
        compiler_params=pltpu.CompilerParams(dimension_semantics=("parallel",)),
    )(x, w0, b0.reshape(1, h), w1, b1.reshape(1, h), gw1)


def _tc_scale(u, dega, degb):
    n, h = u.shape
    nb = _pick_node_block(n)

    def body(u_r, da_r, db_r, o_r):
        o_r[...] = _dinv_block(da_r[...], db_r[...]) * u_r[...]

    row = lambda i: (i, 0)
    return pl.pallas_call(
        body,
        grid=(n // nb,),
        in_specs=[
            pl.BlockSpec((nb, h), row),
            pl.BlockSpec((nb, 128), row), pl.BlockSpec((nb, 128), row),
        ],
        out_specs=pl.BlockSpec((nb, h), row),
        out_shape=jax.ShapeDtypeStruct((n, h), jnp.float32),
        compiler_params=pltpu.CompilerParams(dimension_semantics=("parallel",)),
    )(u, dega, degb)


def _tc_layer(agga, aggb, g_prev, b_prev, w_next, dega, degb):
    n, h = g_prev.shape
    nb = _pick_node_block(n)

    def body(aa_r, ab_r, g_r, b_r, w_r, da_r, db_r, o_r):
        dinv = _dinv_block(da_r[...], db_r[...])
        hcur = jnp.maximum(dinv * (aa_r[...] + ab_r[...] + g_r[...]) + b_r[...], 0.0)
        u = jnp.dot(hcur, w_r[...], preferred_element_type=jnp.float32)
        o_r[...] = dinv * u

    full = lambda i: (0, 0)
    row = lambda i: (i, 0)
    return pl.pallas_call(
        body,
        grid=(n // nb,),
        in_specs=[
            pl.BlockSpec((nb, h), row), pl.BlockSpec((nb, h), row),
            pl.BlockSpec((nb, h), row), pl.BlockSpec((1, h), full),
            pl.BlockSpec((h, h), full),
            pl.BlockSpec((nb, 128), row), pl.BlockSpec((nb, 128), row),
        ],
        out_specs=pl.BlockSpec((nb, h), row),
        out_shape=jax.ShapeDtypeStruct((n, h), jnp.float32),
        compiler_params=pltpu.CompilerParams(dimension_semantics=("parallel",)),
    )(agga, aggb, g_prev, b_prev.reshape(1, h), w_next, dega, degb)


def _tc_readout(agga, aggb, g3, gb3, dega, degb, l1w, l1b, l2row, l2b, batch, num_graphs):
    n, h = g3.shape
    h2 = l1w.shape[1]
    nb = _pick_node_block(n)
    grid = n // nb
    batch3 = batch.reshape(grid, 1, nb)

    def body(aa_r, ab_r, g_r, b_r, da_r, db_r, l1w_r, l1b_r, l2_r, l2b_r, bm_r, o_r):
        i = pl.program_id(0)
        dinv = _dinv_block(da_r[...], db_r[...])
        hcur = jnp.maximum(dinv * (aa_r[...] + ab_r[...] + g_r[...]) + b_r[...], 0.0)
        z1 = jnp.maximum(jnp.dot(hcur, l1w_r[...], preferred_element_type=jnp.float32) + l1b_r[...], 0.0)
        z2 = jnp.sum(z1 * l2_r[...], axis=1) + l2b_r[0, 0]
        bm = bm_r[0, 0, :]
        gids = jax.lax.broadcasted_iota(jnp.int32, (num_graphs, nb), 0)
        onehot = (gids == bm[None, :]).astype(jnp.float32)
        seg = jnp.sum(onehot * z2[None, :], axis=1)

        @pl.when(i == 0)
        def _():
            o_r[...] = jnp.zeros_like(o_r)

        o_r[...] = o_r[...] + seg[:, None]

    full = lambda i: (0, 0)
    row = lambda i: (i, 0)
    return pl.pallas_call(
        body,
        grid=(grid,),
        in_specs=[
            pl.BlockSpec((nb, h), row), pl.BlockSpec((nb, h), row),
            pl.BlockSpec((nb, h), row), pl.BlockSpec((1, h), full),
            pl.BlockSpec((nb, 128), row), pl.BlockSpec((nb, 128), row),
            pl.BlockSpec((h, h2), full), pl.BlockSpec((1, h2), full),
            pl.BlockSpec((1, h2), full), pl.BlockSpec((1, 1), full),
            pl.BlockSpec((1, 1, nb), lambda i: (i, 0, 0)),
        ],
        out_specs=pl.BlockSpec((num_graphs, 128), full),
        out_shape=jax.ShapeDtypeStruct((num_graphs, 128), jnp.float32),
        compiler_params=pltpu.CompilerParams(dimension_semantics=("arbitrary",)),
    )(agga, aggb, g3, gb3.reshape(1, h), dega, degb, l1w, l1b.reshape(1, h2),
      l2row, l2b.reshape(1, 1), batch3)


def kernel(x, pos, edge_index, batch, W0, b0, W1, b1, gW1, gb1, gW2, gb2, gW3, gb3, l1W, l1b, l2W, l2b):
    n, h = x.shape
    e = edge_index.shape[1]
    num_graphs = 64

    src = edge_index[0].astype(jnp.int32)
    dst = edge_index[1].astype(jnp.int32)
    w, pad = _pick_edge_window(e)
    npad = 16 if pad else 0
    if pad:
        src = jnp.concatenate([src, jnp.full((pad,), n, jnp.int32)])
        dst_deg = jnp.concatenate([dst, jnp.full((pad,), n, jnp.int32)])
        dst = jnp.concatenate([dst, jnp.zeros((pad,), jnp.int32)])
    else:
        dst_deg = dst

    ones_w = jnp.ones((_pick_deg_window(e + pad), 128), jnp.float32)
    zeros_deg = jnp.zeros((n + npad, 128), jnp.float32)
    zeros_acc = jnp.zeros((n, h), jnp.float32)
    zrows = jnp.zeros((npad, h), jnp.float32)

    deg_parts = _sc_degree(dst_deg, ones_w, zeros_deg)
    dega = deg_parts[0, :n, :]
    degb = deg_parts[1, :n, :]

    u1 = _tc_k0(x, W0, b0, W1, b1, gW1)
    g = _tc_scale(u1, dega, degb)
    for b_prev, w_next in ((gb1, gW2), (gb2, gW3)):
        gf = jnp.concatenate([g, zrows], axis=0) if pad else g
        agg = _sc_edge_agg(gf, src, dst, zeros_acc)
        g = _tc_layer(agg[0], agg[1], g, b_prev, w_next, dega, degb)

    gf = jnp.concatenate([g, zrows], axis=0) if pad else g
    agg = _sc_edge_agg(gf, src, dst, zeros_acc)
    out = _tc_readout(agg[0], agg[1], g, gb3, dega, degb, l1W, l1b,
                      l2W.reshape(1, -1), l2b, batch.astype(jnp.int32), num_graphs)
    return out[:, :1]


# speedup vs baseline: 1.0941x; 1.0941x over previous
import functools

import jax
import jax.numpy as jnp
from jax.experimental import pallas as pl
from jax.experimental.pallas import tpu as pltpu
from jax.experimental.pallas import tpu_sc as plsc

_NC = 2
_NS = 16


def _pick_edge_window(e):
    for w in (80, 64, 40, 32, 16, 8):
        if e % (_NC * _NS * w) == 0:
            return w, 0
    w = 64
    epad = ((e + _NC * _NS * w - 1) // (_NC * _NS * w)) * (_NC * _NS * w)
    return w, epad - e


def _pick_deg_window(e_padded):
    for w in (200, 160, 128, 80, 64, 40, 32, 16, 8):
        if e_padded % (_NC * _NS * w) == 0:
            return w
    return 8


def _zero_split(nacc):
    for k in range(_NS, 0, -1):
        if nacc % k == 0 and (nacc // k) % 8 == 0:
            return k, nacc // k
    return 1, nacc


_NBUF = 3


def _sc_degree(dst, ones_w, zeros_acc):
    (e,) = dst.shape
    nacc = zeros_acc.shape[0]
    w = ones_w.shape[0]
    chunk = e // (_NC * _NS)
    nwin = chunk // w
    assert nwin >= _NBUF
    nz, rows_per = _zero_split(nacc)

    ni = 6
    assert nwin >= ni

    @functools.partial(
        pl.kernel,
        out_type=jax.ShapeDtypeStruct((_NC, nacc, 128), jnp.float32),
        mesh=plsc.VectorSubcoreMesh(core_axis_name="c", subcore_axis_name="s"),
        scratch_types=[
            [pltpu.VMEM((w,), jnp.int32) for _ in range(ni)],
            pltpu.VMEM((w, 128), jnp.float32),
            pltpu.VMEM_SHARED((nacc, 128), jnp.float32),
            [pltpu.SemaphoreType.DMA for _ in range(ni)],
            [pltpu.SemaphoreType.DMA for _ in range(ni)],
        ],
    )
    def deg_kernel(dst_hbm, ones_hbm, z_hbm, out_hbm, dbuf, ones_v, acc, isem, ssem):
        cid = jax.lax.axis_index("c")
        sid = jax.lax.axis_index("s")
        wid = sid * _NC + cid
        pltpu.sync_copy(ones_hbm, ones_v)

        @pl.when(sid < nz)
        def _():
            pltpu.sync_copy(
                z_hbm.at[pl.ds(sid * rows_per, rows_per)],
                acc.at[pl.ds(sid * rows_per, rows_per)],
            )

        plsc.subcore_barrier()
        base0 = wid * chunk

        def issue_idx(j, b):
            pltpu.async_copy(dst_hbm.at[pl.ds(base0 + j * w, w)], dbuf[b], isem[b])

        def wait_idx(j, b):
            pltpu.make_async_copy(dst_hbm.at[pl.ds(base0 + j * w, w)], dbuf[b], isem[b]).wait()

        def issue_scatter(b):
            pltpu.async_copy(ones_v, acc.at[dbuf[b]], ssem[b], add=True)

        def wait_scatter(b):
            pltpu.make_async_copy(ones_v, acc.at[dbuf[b]], ssem[b]).wait()

        for b in range(ni - 1):
            issue_idx(b, b)

        nouter = -(-nwin // ni) * ni

        @pl.loop(0, nouter, step=ni)
        def _(j0):
            for u in range(ni):
                j = j0 + u

                @pl.when(j < nwin)
                def _():
                    wait_idx(j, u)
                    issue_scatter(u)

                @pl.when(j + ni - 1 < nwin)
                def _():
                    u5 = (u + ni - 1) % ni

                    @pl.when(j >= 1)
                    def _():
                        wait_scatter(u5)

                    issue_idx(j + ni - 1, u5)

        for k in range(nwin - ni, nwin):
            wait_scatter(k % ni)

        plsc.subcore_barrier()

        @pl.when(sid < nz)
        def _():
            pltpu.sync_copy(
                acc.at[pl.ds(sid * rows_per, rows_per)],
                out_hbm.at[cid, pl.ds(sid * rows_per, rows_per)],
            )

    return deg_kernel(dst, ones_w, zeros_acc)


def _sc_edge_agg(g, src, dst, zeros_acc):
    n, h = zeros_acc.shape
    (e,) = src.shape
    w, _ = _pick_edge_window(e)
    chunk = e // (_NC * _NS)
    nwin = chunk // w
    nz, rows_per = _zero_split(n)
    half = n // 2

    ni = 2 * _NBUF
    assert nwin >= ni

    @functools.partial(
        pl.kernel,
        out_type=jax.ShapeDtypeStruct((_NC, n, h), jnp.float32),
        mesh=plsc.VectorSubcoreMesh(core_axis_name="c", subcore_axis_name="s"),
        scratch_types=[
            [pltpu.VMEM((w,), jnp.int32) for _ in range(ni)],
            [pltpu.VMEM((w,), jnp.int32) for _ in range(ni)],
            [pltpu.VMEM((w, h), jnp.float32) for _ in range(_NBUF)],
            pltpu.VMEM_SHARED((n, h), jnp.float32),
            [pltpu.SemaphoreType.DMA for _ in range(ni)],
            [pltpu.SemaphoreType.DMA for _ in range(_NBUF)],
            [pltpu.SemaphoreType.DMA for _ in range(_NBUF)],
        ],
    )
    def agg_kernel(g_hbm, src_hbm, dst_hbm, z_hbm, out_hbm, sbuf, dbuf, rbuf, acc, isem, gsem, ssem):
        cid = jax.lax.axis_index("c")
        sid = jax.lax.axis_index("s")
        wid = sid * _NC + cid

        @pl.when(sid < nz)
        def _():
            pltpu.sync_copy(
                z_hbm.at[pl.ds(sid * rows_per, rows_per)],
                acc.at[pl.ds(sid * rows_per, rows_per)],
            )

        plsc.subcore_barrier()
        base0 = wid * chunk

        def issue_idx(j, b):
            pltpu.async_copy(src_hbm.at[pl.ds(base0 + j * w, w)], sbuf[b], isem[b])
            pltpu.async_copy(dst_hbm.at[pl.ds(base0 + j * w, w)], dbuf[b], isem[b])

        def wait_idx(j, b):
            pltpu.make_async_copy(src_hbm.at[pl.ds(base0 + j * w, w)], sbuf[b], isem[b]).wait()
            pltpu.make_async_copy(dst_hbm.at[pl.ds(base0 + j * w, w)], dbuf[b], isem[b]).wait()

        def issue_gather(bi, br):
            pltpu.async_copy(g_hbm.at[sbuf[bi]], rbuf[br], gsem[br])

        def wait_gather(bi, br):
            pltpu.make_async_copy(g_hbm.at[sbuf[bi]], rbuf[br], gsem[br]).wait()

        def issue_scatter(bi, br):
            pltpu.async_copy(rbuf[br], acc.at[dbuf[bi]], ssem[br], add=True)

        def wait_scatter(bi, br):
            pltpu.make_async_copy(rbuf[br], acc.at[dbuf[bi]], ssem[br]).wait()

        lead = ni - 3
        for b in range(lead):
            issue_idx(b, b)
        for b in range(2):
            wait_idx(b, b)
            issue_gather(b, b)

        nouter = -(-nwin // ni) * ni

        @pl.loop(0, nouter, step=ni)
        def _(j0):
            for u in range(ni):
                j = j0 + u
                br = u % _NBUF

                @pl.when(j < nwin)
                def _():
                    wait_gather(u, br)
                    issue_scatter(u, br)

                @pl.when(j + 2 < nwin)
                def _():
                    u2 = (u + 2) % ni

                    @pl.when(j >= _NBUF - 2)
                    def _():
                        wait_scatter((u + ni + 2 - _NBUF) % ni, (u + 2) % _NBUF)

                    wait_idx(j + 2, u2)
                    issue_gather(u2, (u + 2) % _NBUF)

                @pl.when(j + lead < nwin)
                def _():
                    issue_idx(j + lead, (u + lead) % ni)

        for k in range(nwin - _NBUF, nwin):
            wait_scatter(k % ni, k % _NBUF)

        plsc.subcore_barrier()

        @pl.when(sid < nz)
        def _():
            pltpu.sync_copy(
                acc.at[pl.ds(sid * rows_per, rows_per)],
                out_hbm.at[cid, pl.ds(sid * rows_per, rows_per)],
            )

    return agg_kernel(g, src, dst, zeros_acc)


def _dinv_block(da, db):
    return jax.lax.rsqrt(1.0 + da[:, :1] + db[:, :1])


def _pick_node_block(n):
    for nb in (1000, 2000, 500, 1250, 200, 8):
        if n % nb == 0 and nb % 8 == 0:
            return nb
    return n


def _tc_k0(x, w0, b0, w1, b1, gw1):
    n, h = x.shape
    nb = _pick_node_block(n)

    def body(x_r, w0_r, b0_r, w1_r, b1_r, gw_r, o_r):
        a = jnp.maximum(jnp.dot(x_r[...], w0_r[...], preferred_element_type=jnp.float32) + b0_r[...], 0.0)
        h0 = jnp.dot(a, w1_r[...], preferred_element_type=jnp.float32) + b1_r[...]
        o_r[...] = jnp.dot(h0, gw_r[...], preferred_element_type=jnp.float32)

    full = lambda i: (0, 0)
    row = lambda i: (i, 0)
    return pl.pallas_call(
        body,
        grid=(n // nb,),
        in_specs=[
            pl.BlockSpec((nb, h), row),
            pl.BlockSpec((h, h), full), pl.BlockSpec((1, h), full),
            pl.BlockSpec((h, h), full), pl.BlockSpec((1, h), full),
            pl.BlockSpec((h, h), full),
        ],
        out_specs=pl.BlockSpec((nb, h), row),
        out_shape=jax.ShapeDtypeStruct((n, h), jnp.float32),
        compiler_params=pltpu.CompilerParams(dimension_semantics=("parallel",)),
    )(x, w0, b0.reshape(1, h), w1, b1.reshape(1, h), gw1)


def _tc_scale(u, dega, degb):
    n, h = u.shape
    nb = _pick_node_block(n)

    def body(u_r, da_r, db_r, o_r):
        o_r[...] = _dinv_block(da_r[...], db_r[...]) * u_r[...]

    row = lambda i: (i, 0)
    return pl.pallas_call(
        body,
        grid=(n // nb,),
        in_specs=[
            pl.BlockSpec((nb, h), row),
            pl.BlockSpec((nb, 128), row), pl.BlockSpec((nb, 128), row),
        ],
        out_specs=pl.BlockSpec((nb, h), row),
        out_shape=jax.ShapeDtypeStruct((n, h), jnp.float32),
        compiler_params=pltpu.CompilerParams(dimension_semantics=("parallel",)),
    )(u, dega, degb)


def _tc_layer(agga, aggb, g_prev, b_prev, w_next, dega, degb):
    n, h = g_prev.shape
    nb = _pick_node_block(n)

    def body(aa_r, ab_r, g_r, b_r, w_r, da_r, db_r, o_r):
        dinv = _dinv_block(da_r[...], db_r[...])
        hcur = jnp.maximum(dinv * (aa_r[...] + ab_r[...] + g_r[...]) + b_r[...], 0.0)
        u = jnp.dot(hcur, w_r[...], preferred_element_type=jnp.float32)
        o_r[...] = dinv * u

    full = lambda i: (0, 0)
    row = lambda i: (i, 0)
    return pl.pallas_call(
        body,
        grid=(n // nb,),
        in_specs=[
            pl.BlockSpec((nb, h), row), pl.BlockSpec((nb, h), row),
            pl.BlockSpec((nb, h), row), pl.BlockSpec((1, h), full),
            pl.BlockSpec((h, h), full),
            pl.BlockSpec((nb, 128), row), pl.BlockSpec((nb, 128), row),
        ],
        out_specs=pl.BlockSpec((nb, h), row),
        out_shape=jax.ShapeDtypeStruct((n, h), jnp.float32),
        compiler_params=pltpu.CompilerParams(dimension_semantics=("parallel",)),
    )(agga, aggb, g_prev, b_prev.reshape(1, h), w_next, dega, degb)


def _tc_readout(agga, aggb, g3, gb3, dega, degb, l1w, l1b, l2row, l2b, batch, num_graphs):
    n, h = g3.shape
    h2 = l1w.shape[1]
    nb = _pick_node_block(n)
    grid = n // nb
    batch3 = batch.reshape(grid, 1, nb)

    def body(aa_r, ab_r, g_r, b_r, da_r, db_r, l1w_r, l1b_r, l2_r, l2b_r, bm_r, o_r):
        i = pl.program_id(0)
        dinv = _dinv_block(da_r[...], db_r[...])
        hcur = jnp.maximum(dinv * (aa_r[...] + ab_r[...] + g_r[...]) + b_r[...], 0.0)
        z1 = jnp.maximum(jnp.dot(hcur, l1w_r[...], preferred_element_type=jnp.float32) + l1b_r[...], 0.0)
        z2 = jnp.sum(z1 * l2_r[...], axis=1) + l2b_r[0, 0]
        bm = bm_r[0, 0, :]
        gids = jax.lax.broadcasted_iota(jnp.int32, (num_graphs, nb), 0)
        onehot = (gids == bm[None, :]).astype(jnp.float32)
        seg = jnp.sum(onehot * z2[None, :], axis=1)

        @pl.when(i == 0)
        def _():
            o_r[...] = jnp.zeros_like(o_r)

        o_r[...] = o_r[...] + seg[:, None]

    full = lambda i: (0, 0)
    row = lambda i: (i, 0)
    return pl.pallas_call(
        body,
        grid=(grid,),
        in_specs=[
            pl.BlockSpec((nb, h), row), pl.BlockSpec((nb, h), row),
            pl.BlockSpec((nb, h), row), pl.BlockSpec((1, h), full),
            pl.BlockSpec((nb, 128), row), pl.BlockSpec((nb, 128), row),
            pl.BlockSpec((h, h2), full), pl.BlockSpec((1, h2), full),
            pl.BlockSpec((1, h2), full), pl.BlockSpec((1, 1), full),
            pl.BlockSpec((1, 1, nb), lambda i: (i, 0, 0)),
        ],
        out_specs=pl.BlockSpec((num_graphs, 128), full),
        out_shape=jax.ShapeDtypeStruct((num_graphs, 128), jnp.float32),
        compiler_params=pltpu.CompilerParams(dimension_semantics=("arbitrary",)),
    )(agga, aggb, g3, gb3.reshape(1, h), dega, degb, l1w, l1b.reshape(1, h2),
      l2row, l2b.reshape(1, 1), batch3)


def kernel(x, pos, edge_index, batch, W0, b0, W1, b1, gW1, gb1, gW2, gb2, gW3, gb3, l1W, l1b, l2W, l2b):
    n, h = x.shape
    e = edge_index.shape[1]
    num_graphs = 64

    src = edge_index[0].astype(jnp.int32)
    dst = edge_index[1].astype(jnp.int32)
    w, pad = _pick_edge_window(e)
    npad = 16 if pad else 0
    if pad:
        src = jnp.concatenate([src, jnp.full((pad,), n, jnp.int32)])
        dst_deg = jnp.concatenate([dst, jnp.full((pad,), n, jnp.int32)])
        dst = jnp.concatenate([dst, jnp.zeros((pad,), jnp.int32)])
    else:
        dst_deg = dst

    ones_w = jnp.ones((_pick_deg_window(e + pad), 128), jnp.float32)
    zeros_deg = jnp.zeros((n + npad, 128), jnp.float32)
    zeros_acc = jnp.zeros((n, h), jnp.float32)
    zrows = jnp.zeros((npad, h), jnp.float32)

    deg_parts = _sc_degree(dst_deg, ones_w, zeros_deg)
    dega = deg_parts[0, :n, :]
    degb = deg_parts[1, :n, :]

    g = _tc_scale(_tc_k0(x, W0, b0, W1, b1, gW1), dega, degb)
    for b_prev, w_next in ((gb1, gW2), (gb2, gW3)):
        gf = jnp.concatenate([g, zrows], axis=0) if pad else g
        agg = _sc_edge_agg(gf, src, dst, zeros_acc)
        g = _tc_layer(agg[0], agg[1], g, b_prev, w_next, dega, degb)

    gf = jnp.concatenate([g, zrows], axis=0) if pad else g
    agg = _sc_edge_agg(gf, src, dst, zeros_acc)
    out = _tc_readout(agg[0], agg[1], g, gb3, dega, degb, l1W, l1b,
                      l2W.reshape(1, -1), l2b, batch.astype(jnp.int32), num_graphs)
    return out[:, :1]
